# baseline (device time: 26816 ns/iter reference)
import math

import jax
import jax.numpy as jnp
from jax import lax
from jax.experimental import pallas as pl
from jax.experimental.pallas import tpu as pltpu

N_DEV = 4
N_CHUNK = 4


def kernel(q, k, v):
    S, D = q.shape
    R = S // N_CHUNK

    def body(q_ref, k_ref, v_ref, out_ref, comm_ref, send_sems, recv_sems):
        my = lax.axis_index("i")
        left = (my + N_DEV - 1) % N_DEV
        right = (my + 1) % N_DEV

        barrier_sem = pltpu.get_barrier_semaphore()
        for nbr in (left, right):
            pl.semaphore_signal(
                barrier_sem, inc=1,
                device_id=(nbr,), device_id_type=pl.DeviceIdType.MESH,
            )
        pl.semaphore_wait(barrier_sem, 2)

        scale = 1.0 / math.sqrt(D)
        q_s = (q_ref[...] * scale).astype(jnp.bfloat16)
        comm_ref[0, :, :D] = k_ref[...].astype(jnp.bfloat16)
        comm_ref[0, :, D:] = v_ref[...].astype(jnp.bfloat16)

        def chunk_rdma(h, c):
            return pltpu.make_async_remote_copy(
                src_ref=comm_ref.at[h, pl.ds(c * R, R), :],
                dst_ref=comm_ref.at[h + 1, pl.ds(c * R, R), :],
                send_sem=send_sems.at[h, c],
                recv_sem=recv_sems.at[h, c],
                device_id=(right,),
                device_id_type=pl.DeviceIdType.MESH,
            )

        l = jnp.zeros((S, 1), jnp.float32)
        acc = jnp.zeros((S, D), jnp.float32)
        ones_col = jnp.ones((R, 1), jnp.bfloat16)

        def accumulate_chunk(slot, c, l, acc):
            kv = comm_ref[slot, pl.ds(c * R, R), :]
            kb = kv[:, :D]
            vb = kv[:, D:]
            s = lax.dot_general(
                q_s, kb, (((1,), (1,)), ((), ())),
                preferred_element_type=jnp.float32,
            )
            p = jnp.exp(s).astype(jnp.bfloat16)
            acc = acc + lax.dot_general(
                p, vb, (((1,), (0,)), ((), ())),
                preferred_element_type=jnp.float32,
            )
            l = l + lax.dot_general(
                p, ones_col, (((1,), (0,)), ((), ())),
                preferred_element_type=jnp.float32,
            )
            return l, acc

        for c in range(N_CHUNK):
            chunk_rdma(0, c).start()
        for c in range(N_CHUNK):
            l, acc = accumulate_chunk(0, c, l, acc)

        for slot in range(1, N_DEV):
            for c in range(N_CHUNK):
                chunk_rdma(slot - 1, c).wait_recv()
                if slot < N_DEV - 1:
                    chunk_rdma(slot, c).start()
                l, acc = accumulate_chunk(slot, c, l, acc)

        out_ref[...] = acc / l

        for h in range(N_DEV - 1):
            for c in range(N_CHUNK):
                chunk_rdma(h, c).wait_send()

    return pl.pallas_call(
        body,
        out_shape=jax.ShapeDtypeStruct((S, D), jnp.float32),
        in_specs=[pl.BlockSpec(memory_space=pltpu.VMEM)] * 3,
        out_specs=pl.BlockSpec(memory_space=pltpu.VMEM),
        scratch_shapes=[
            pltpu.VMEM((N_DEV, S, 2 * D), jnp.bfloat16),
            pltpu.SemaphoreType.DMA((N_DEV - 1, N_CHUNK)),
            pltpu.SemaphoreType.DMA((N_DEV - 1, N_CHUNK)),
        ],
        compiler_params=pltpu.CompilerParams(collective_id=0),
    )(q, k, v)


# device time: 18786 ns/iter; 1.4274x vs baseline; 1.4274x over previous
import math

import jax
import jax.numpy as jnp
from jax import lax
from jax.experimental import pallas as pl
from jax.experimental.pallas import tpu as pltpu

N_DEV = 4
N_CHUNK = 4

OWN, L, R, FAR = 0, 1, 2, 3
R_ORDER = (0, 1, 2, 3)
L_ORDER = (2, 3, 0, 1)


def kernel(q, k, v):
    S, D = q.shape
    RC = S // N_CHUNK

    def body(q_ref, k_ref, v_ref, out_ref, comm_ref, send_sems, recv_sems):
        my = lax.axis_index("i")
        left = (my + N_DEV - 1) % N_DEV
        right = (my + 1) % N_DEV

        barrier_sem = pltpu.get_barrier_semaphore()
        for nbr in (left, right):
            pl.semaphore_signal(
                barrier_sem, inc=1,
                device_id=(nbr,), device_id_type=pl.DeviceIdType.MESH,
            )
        pl.semaphore_wait(barrier_sem, 2)

        scale = 1.0 / math.sqrt(D)
        q_s = (q_ref[...] * scale).astype(jnp.bfloat16)
        comm_ref[OWN, :, :D] = k_ref[...].astype(jnp.bfloat16)
        comm_ref[OWN, :, D:] = v_ref[...].astype(jnp.bfloat16)

        def rdma(src_slot, dst_slot, c, sem_row, target):
            return pltpu.make_async_remote_copy(
                src_ref=comm_ref.at[src_slot, pl.ds(c * RC, RC), :],
                dst_ref=comm_ref.at[dst_slot, pl.ds(c * RC, RC), :],
                send_sem=send_sems.at[sem_row, c],
                recv_sem=recv_sems.at[sem_row, c],
                device_id=(target,),
                device_id_type=pl.DeviceIdType.MESH,
            )

        own_r = lambda c: rdma(OWN, L, c, 0, right)
        own_l = lambda c: rdma(OWN, R, c, 1, left)
        relay_r = lambda c: rdma(L, FAR, c, 2, right)
        relay_l = lambda c: rdma(R, FAR, c, 2, left)

        l_sum = jnp.zeros((S, 1), jnp.float32)
        acc = jnp.zeros((S, D), jnp.float32)

        def accumulate_chunk(slot, c, l_sum, acc):
            kv = comm_ref[slot, pl.ds(c * RC, RC), :]
            kb = kv[:, :D]
            vb = kv[:, D:]
            s = lax.dot_general(
                q_s, kb, (((1,), (1,)), ((), ())),
                preferred_element_type=jnp.float32,
            )
            p = jnp.exp(s)
            l_sum = l_sum + jnp.sum(p, axis=1, keepdims=True)
            acc = acc + lax.dot_general(
                p.astype(jnp.bfloat16), vb, (((1,), (0,)), ((), ())),
                preferred_element_type=jnp.float32,
            )
            return l_sum, acc

        for c in R_ORDER:
            own_r(c).start()
        for c in L_ORDER:
            own_l(c).start()

        for c in range(N_CHUNK):
            l_sum, acc = accumulate_chunk(OWN, c, l_sum, acc)

        own_r(0).wait_recv()
        relay_r(0).start()
        own_l(2).wait_recv()
        relay_l(2).start()
        own_r(1).wait_recv()
        relay_r(1).start()
        own_l(3).wait_recv()
        relay_l(3).start()

        for slot, c in ((L, 0), (L, 1), (R, 2), (R, 3)):
            l_sum, acc = accumulate_chunk(slot, c, l_sum, acc)
        for flow, slot, c in ((own_r, L, 2), (own_r, L, 3),
                              (own_l, R, 0), (own_l, R, 1)):
            flow(c).wait_recv()
            l_sum, acc = accumulate_chunk(slot, c, l_sum, acc)

        for c, relay in ((0, relay_r), (2, relay_l), (1, relay_r), (3, relay_l)):
            relay(c).wait_recv()
            l_sum, acc = accumulate_chunk(FAR, c, l_sum, acc)

        out_ref[...] = acc / l_sum

        for c in range(N_CHUNK):
            own_r(c).wait_send()
            own_l(c).wait_send()
        for c in (0, 1):
            relay_r(c).wait_send()
        for c in (2, 3):
            relay_l(c).wait_send()

    return pl.pallas_call(
        body,
        out_shape=jax.ShapeDtypeStruct((S, D), jnp.float32),
        in_specs=[pl.BlockSpec(memory_space=pltpu.VMEM)] * 3,
        out_specs=pl.BlockSpec(memory_space=pltpu.VMEM),
        scratch_shapes=[
            pltpu.VMEM((4, S, 2 * D), jnp.bfloat16),
            pltpu.SemaphoreType.DMA((3, N_CHUNK)),
            pltpu.SemaphoreType.DMA((3, N_CHUNK)),
        ],
        compiler_params=pltpu.CompilerParams(collective_id=0),
    )(q, k, v)
